# Initial kernel scaffold; baseline (speedup 1.0000x reference)
import jax
import jax.numpy as jnp
from jax.experimental import pallas as pl


def kernel(x, Wg, W1, b1, W2, b2):
    raise NotImplementedError("write your pallas kernel here")


# speedup vs baseline: 1.2700x; 1.2700x over previous
import functools

import jax
import jax.numpy as jnp
from jax import lax
from jax.experimental import pallas as pl
from jax.experimental.pallas import tpu as pltpu
from jax.experimental.pallas import tpu_sc as plsc

E = 8
D = 768
H = 3072
T = 2048
TILE = 256
NT = 23
NA = NT * TILE
NW = 32
TPW = T // NW
CB = 512
NB = T // CB


def _router_body(x_ref, wg_ref, dest_ref, gates_ref, counts_ref, aux_ref):
    x = x_ref[...]
    logits = jnp.dot(x, wg_ref[...], preferred_element_type=jnp.float32)
    m = jnp.max(logits, axis=-1, keepdims=True)
    ex = jnp.exp(logits - m)
    probs = ex / jnp.sum(ex, axis=-1, keepdims=True)

    lane = lax.broadcasted_iota(jnp.int32, (T, E), 1)
    v1 = jnp.max(probs, axis=-1, keepdims=True)
    i1 = jnp.min(jnp.where(probs == v1, lane, E), axis=-1, keepdims=True)
    m0 = (lane == i1).astype(jnp.float32)
    probs2 = jnp.where(m0 > 0.0, -1.0, probs)
    v2 = jnp.max(probs2, axis=-1, keepdims=True)
    i2 = jnp.min(jnp.where(probs2 == v2, lane, E), axis=-1, keepdims=True)
    m1 = (lane == i2).astype(jnp.float32)

    denom = v1 + v2
    gates_ref[...] = jnp.concatenate([v1 / denom, v2 / denom], axis=-1)

    mm = m0 + m1
    r = lax.broadcasted_iota(jnp.int32, (CB, CB), 0)
    c = lax.broadcasted_iota(jnp.int32, (CB, CB), 1)
    ls = (r > c).astype(jnp.float32)
    blocks = []
    off = jnp.zeros((1, E), jnp.float32)
    for b in range(NB):
        mb = mm[b * CB:(b + 1) * CB, :]
        blocks.append(jnp.dot(ls, mb, preferred_element_type=jnp.float32) + off)
        off = off + jnp.sum(mb, axis=0, keepdims=True)
    csum = jnp.concatenate(blocks, axis=0)
    counts = off

    aligned = jnp.floor((counts + float(TILE - 1)) * (1.0 / TILE)) * float(TILE)
    er = lax.broadcasted_iota(jnp.int32, (E, E), 0)
    ec = lax.broadcasted_iota(jnp.int32, (E, E), 1)
    u = (er < ec).astype(jnp.float32)
    astart = jnp.dot(aligned, u, preferred_element_type=jnp.float32)

    p0 = jnp.sum(csum * m0, axis=-1, keepdims=True)
    p1 = jnp.sum(csum * m1, axis=-1, keepdims=True)
    a0 = jnp.sum(astart * m0, axis=-1, keepdims=True)
    a1 = jnp.sum(astart * m1, axis=-1, keepdims=True)
    dest_ref[...] = jnp.concatenate(
        [(a0 + p0).astype(jnp.int32), (a1 + p1).astype(jnp.int32)], axis=-1)
    counts_ref[...] = counts.astype(jnp.int32)

    imp = jnp.mean(probs, axis=0, keepdims=True)
    load = jnp.mean(m0, axis=0, keepdims=True)
    aux_ref[...] = jnp.sum(imp * load, axis=-1, keepdims=True) * (E * 0.01)


def _router(x2, wg):
    return pl.pallas_call(
        _router_body,
        out_shape=(
            jax.ShapeDtypeStruct((T, 2), jnp.int32),
            jax.ShapeDtypeStruct((T, 2), jnp.float32),
            jax.ShapeDtypeStruct((1, E), jnp.int32),
            jax.ShapeDtypeStruct((1, 1), jnp.float32),
        ),
    )(x2, wg)


def _sc_scatter_body(x_hbm, d0_hbm, d1_hbm, xs_hbm, idx0_v, idx1_v, rows_v,
                     sem0, sem1):
    wid = lax.axis_index("s") * 2 + lax.axis_index("c")
    base = wid * TPW
    pltpu.sync_copy(d0_hbm.at[wid], idx0_v)
    pltpu.sync_copy(d1_hbm.at[wid], idx1_v)
    pltpu.sync_copy(x_hbm.at[pl.ds(base, TPW)], rows_v)
    c0 = pltpu.async_copy(rows_v, xs_hbm.at[idx0_v], sem0)
    c1 = pltpu.async_copy(rows_v, xs_hbm.at[idx1_v], sem1)
    c0.wait()
    c1.wait()


@functools.cache
def _sc_scatter_kernel():
    return functools.partial(
        pl.kernel,
        out_type=jax.ShapeDtypeStruct((NA, D), jnp.float32),
        mesh=plsc.VectorSubcoreMesh(core_axis_name="c", subcore_axis_name="s"),
        scratch_types=[
            pltpu.VMEM((TPW,), jnp.int32),
            pltpu.VMEM((TPW,), jnp.int32),
            pltpu.VMEM((TPW, D), jnp.float32),
            pltpu.SemaphoreType.DMA,
            pltpu.SemaphoreType.DMA,
        ],
    )(_sc_scatter_body)


def _ffn_body(te_ref, tu_ref, xs_ref, w1_ref, b1_ref, w2_ref, b2_ref, out_ref):
    t = pl.program_id(0)

    @pl.when(tu_ref[t] > 0)
    def _():
        xt = xs_ref[...]
        h = jnp.dot(xt, w1_ref[0], preferred_element_type=jnp.float32)
        h = jnp.maximum(h + b1_ref[0], 0.0)
        out_ref[...] = (jnp.dot(h, w2_ref[0], preferred_element_type=jnp.float32)
                        + b2_ref[0])


def _ffn(te, tu, xs, w1, b1, w2, b2):
    grid_spec = pltpu.PrefetchScalarGridSpec(
        num_scalar_prefetch=2,
        grid=(NT,),
        in_specs=[
            pl.BlockSpec((TILE, D), lambda t, te, tu: (t, 0)),
            pl.BlockSpec((1, D, H), lambda t, te, tu: (te[t], 0, 0)),
            pl.BlockSpec((1, 1, H), lambda t, te, tu: (te[t], 0, 0)),
            pl.BlockSpec((1, H, D), lambda t, te, tu: (te[t], 0, 0)),
            pl.BlockSpec((1, 1, D), lambda t, te, tu: (te[t], 0, 0)),
        ],
        out_specs=pl.BlockSpec((TILE, D), lambda t, te, tu: (t, 0)),
    )
    return pl.pallas_call(
        _ffn_body,
        grid_spec=grid_spec,
        out_shape=jax.ShapeDtypeStruct((NA, D), jnp.float32),
    )(te, tu, xs, w1, b1.reshape(E, 1, H), w2, b2.reshape(E, 1, D))


def _sc_gather_body(os_hbm, d0_hbm, d1_hbm, a_hbm, b_hbm, idx0_v, idx1_v,
                    rows0_v, rows1_v, sem0, sem1):
    wid = lax.axis_index("s") * 2 + lax.axis_index("c")
    base = wid * TPW
    pltpu.sync_copy(d0_hbm.at[wid], idx0_v)
    pltpu.sync_copy(d1_hbm.at[wid], idx1_v)
    c0 = pltpu.async_copy(os_hbm.at[idx0_v], rows0_v, sem0)
    c1 = pltpu.async_copy(os_hbm.at[idx1_v], rows1_v, sem1)
    c0.wait()
    pltpu.sync_copy(rows0_v, a_hbm.at[pl.ds(base, TPW)])
    c1.wait()
    pltpu.sync_copy(rows1_v, b_hbm.at[pl.ds(base, TPW)])


@functools.cache
def _sc_gather_kernel():
    return functools.partial(
        pl.kernel,
        out_type=(
            jax.ShapeDtypeStruct((T, D), jnp.float32),
            jax.ShapeDtypeStruct((T, D), jnp.float32),
        ),
        mesh=plsc.VectorSubcoreMesh(core_axis_name="c", subcore_axis_name="s"),
        scratch_types=[
            pltpu.VMEM((TPW,), jnp.int32),
            pltpu.VMEM((TPW,), jnp.int32),
            pltpu.VMEM((TPW, D), jnp.float32),
            pltpu.VMEM((TPW, D), jnp.float32),
            pltpu.SemaphoreType.DMA,
            pltpu.SemaphoreType.DMA,
        ],
    )(_sc_gather_body)


def _combine_body(a_ref, b_ref, g_ref, y_ref):
    g = g_ref[...]
    y_ref[...] = g[:, 0:1] * a_ref[...] + g[:, 1:2] * b_ref[...]


def _combine(a, b, gates):
    return pl.pallas_call(
        _combine_body,
        out_shape=jax.ShapeDtypeStruct((T, D), jnp.float32),
    )(a, b, gates)


def kernel(x, Wg, W1, b1, W2, b2):
    x2 = x.reshape(T, D)
    dest, gates, counts, aux = _router(x2, Wg)

    counts1 = counts.reshape(E)
    aligned = (counts1 + TILE - 1) // TILE * TILE
    ends = jnp.cumsum(aligned)
    tstart = jnp.arange(NT, dtype=jnp.int32) * TILE
    tile_expert = jnp.sum((tstart[:, None] >= ends[None, :]).astype(jnp.int32),
                          axis=1)
    tile_used = (tstart < ends[-1]).astype(jnp.int32)
    last_e = jnp.max(jnp.where(counts1 > 0, jnp.arange(E, dtype=jnp.int32), 0))
    te = jnp.where(tile_used > 0, tile_expert, last_e).astype(jnp.int32)

    d0 = dest[:, 0].reshape(NW, TPW)
    d1 = dest[:, 1].reshape(NW, TPW)

    xs = _sc_scatter_kernel()(x2, d0, d1)
    os = _ffn(te, tile_used, xs, W1, b1, W2, b2)
    a, b = _sc_gather_kernel()(os, d0, d1)
    y = _combine(a, b, gates)
    return y.reshape(1, T, D), aux.reshape(())
